# Initial kernel scaffold; baseline (speedup 1.0000x reference)
import functools

import jax
import jax.numpy as jnp
from jax import lax
from jax.experimental import pallas as pl
from jax.experimental.pallas import tpu as pltpu
from jax.experimental.pallas import tpu_sc as plsc

N = 4608
D = 256
K = 1024
EPSILON = 0.01

BN = 512
BF = 512

_SC_INFO = plsc.get_sparse_core_info()
_NW = _SC_INFO.num_cores * _SC_INFO.num_subcores
_BPW = N // _NW


def _dist_body(z_ref, cb_ref, idx_ref, dmin_ref, ent_ref):
    zb = z_ref[...]
    cb = cb_ref[...]
    g = lax.dot_general(zb, cb, (((1,), (1,)), ((), ())),
                        preferred_element_type=jnp.float32,
                        precision=lax.Precision.HIGHEST)
    zn = jnp.sum(zb * zb, axis=1, keepdims=True)
    cn = jnp.sum(cb * cb, axis=1)
    d2 = zn - 2.0 * g + cn[None, :]
    d = jnp.sqrt(jnp.maximum(d2, 1e-12))

    dmin = jnp.min(d, axis=1)
    iota = lax.broadcasted_iota(jnp.int32, d.shape, 1)
    hit = jnp.where(d == dmin[:, None], iota, K)
    idx_ref[...] = jnp.min(hit, axis=1).astype(jnp.int32)
    dmin_ref[...] = dmin

    x = dmin[:, None] - d
    e = jnp.exp(x)
    s = jnp.sum(e, axis=1)
    sx = jnp.sum(e * x, axis=1)
    ent_ref[...] = jnp.log(s) - sx / s


_dist_call = pl.pallas_call(
    _dist_body,
    grid=(N // BN,),
    in_specs=[
        pl.BlockSpec((BN, D), lambda i: (i, 0)),
        pl.BlockSpec((K, D), lambda i: (0, 0)),
    ],
    out_specs=[
        pl.BlockSpec((BN,), lambda i: (i,)),
        pl.BlockSpec((BN,), lambda i: (i,)),
        pl.BlockSpec((BN,), lambda i: (i,)),
    ],
    out_shape=[
        jax.ShapeDtypeStruct((N,), jnp.int32),
        jax.ShapeDtypeStruct((N,), jnp.float32),
        jax.ShapeDtypeStruct((N,), jnp.float32),
    ],
)


@functools.partial(
    pl.kernel,
    mesh=plsc.VectorSubcoreMesh(core_axis_name="c", subcore_axis_name="s"),
    out_type=jax.ShapeDtypeStruct((N, D), jnp.float32),
    scratch_types=[
        pltpu.VMEM((_BPW,), jnp.int32),
        pltpu.VMEM((_BPW, D), jnp.float32),
        pltpu.SemaphoreType.DMA,
    ],
)
def _sc_gather(cb_hbm, idx_hbm, out_hbm, idx_v, rows_v, sem):
    wid = lax.axis_index("s") * _SC_INFO.num_cores + lax.axis_index("c")
    base = wid * _BPW
    pltpu.sync_copy(idx_hbm.at[pl.ds(base, _BPW)], idx_v)
    pltpu.async_copy(cb_hbm.at[idx_v], rows_v, sem).wait()
    pltpu.sync_copy(rows_v, out_hbm.at[pl.ds(base, _BPW)])


def _fill_body(dmin_ref, ent_ref, q_ref, y_ref, loss_ref, err_ref):
    c = (jnp.sum(dmin_ref[...]) + 0.1 * jnp.sum(ent_ref[...])) / N
    loss_ref[...] = jnp.full((BF, N), c, dtype=jnp.float32)
    diff = q_ref[...] - y_ref[...]
    dev = jnp.sqrt(jnp.sum(diff * diff, axis=1))
    err_ref[...] = (dev > EPSILON).astype(jnp.float32)


_fill_call = pl.pallas_call(
    _fill_body,
    grid=(N // BF,),
    in_specs=[
        pl.BlockSpec((N,), lambda i: (0,)),
        pl.BlockSpec((N,), lambda i: (0,)),
        pl.BlockSpec((BF, D), lambda i: (i, 0)),
        pl.BlockSpec((BF, D), lambda i: (i, 0)),
    ],
    out_specs=[
        pl.BlockSpec((BF, N), lambda i: (i, 0)),
        pl.BlockSpec((BF,), lambda i: (i,)),
    ],
    out_shape=[
        jax.ShapeDtypeStruct((N, N), jnp.float32),
        jax.ShapeDtypeStruct((N,), jnp.float32),
    ],
)


def kernel(z, y_base, codebook, iter_k):
    idx, dmin, ent = _dist_call(z, codebook)
    quantized = _sc_gather(codebook, idx)
    quant_loss, error_times = _fill_call(dmin, ent, quantized, y_base)
    return quantized, quant_loss, error_times


# speedup vs baseline: 1.2685x; 1.2685x over previous
import functools

import jax
import jax.numpy as jnp
from jax import lax
from jax.experimental import pallas as pl
from jax.experimental.pallas import tpu as pltpu
from jax.experimental.pallas import tpu_sc as plsc

N = 4608
D = 256
K = 1024
EPSILON = 0.01

BN = 512
BF = 512

_SC_CORES = 2
_SC_SUBCORES = 16
_NW = _SC_CORES * _SC_SUBCORES
_BPW = N // _NW


def _dist_body(z_ref, cb_ref, idx_ref, dmin_ref, ent_ref):
    zb = z_ref[...]
    cb = cb_ref[...]
    g = lax.dot_general(zb, cb, (((1,), (1,)), ((), ())),
                        preferred_element_type=jnp.float32)
    zn = jnp.sum(zb * zb, axis=1, keepdims=True)
    cn = jnp.sum(cb * cb, axis=1)
    d2 = zn - 2.0 * g + cn[None, :]
    d = jnp.sqrt(jnp.maximum(d2, 1e-12))

    dmin = jnp.min(d, axis=1)
    iota = lax.broadcasted_iota(jnp.int32, d.shape, 1)
    hit = jnp.where(d == dmin[:, None], iota, K)
    idx_ref[...] = jnp.min(hit, axis=1).astype(jnp.int32)
    dmin_ref[...] = dmin

    x = dmin[:, None] - d
    e = jnp.exp(x)
    s = jnp.sum(e, axis=1)
    sx = jnp.sum(e * x, axis=1)
    ent_ref[...] = jnp.log(s) - sx / s


_dist_call = pl.pallas_call(
    _dist_body,
    grid=(N // BN,),
    in_specs=[
        pl.BlockSpec((BN, D), lambda i: (i, 0)),
        pl.BlockSpec((K, D), lambda i: (0, 0)),
    ],
    out_specs=[
        pl.BlockSpec((BN,), lambda i: (i,)),
        pl.BlockSpec((BN,), lambda i: (i,)),
        pl.BlockSpec((BN,), lambda i: (i,)),
    ],
    out_shape=[
        jax.ShapeDtypeStruct((N,), jnp.int32),
        jax.ShapeDtypeStruct((N,), jnp.float32),
        jax.ShapeDtypeStruct((N,), jnp.float32),
    ],
)


@functools.cache
def _sc_gather_call():
    @functools.partial(
        pl.kernel,
        mesh=plsc.VectorSubcoreMesh(core_axis_name="c", subcore_axis_name="s"),
        out_type=jax.ShapeDtypeStruct((N, D), jnp.float32),
        scratch_types=[
            pltpu.VMEM((_BPW,), jnp.int32),
            pltpu.VMEM((_BPW, D), jnp.float32),
            pltpu.SemaphoreType.DMA,
        ],
    )
    def _sc_gather(cb_hbm, idx_hbm, out_hbm, idx_v, rows_v, sem):
        wid = lax.axis_index("s") * _SC_CORES + lax.axis_index("c")
        base = wid * _BPW
        pltpu.sync_copy(idx_hbm.at[pl.ds(base, _BPW)], idx_v)
        pltpu.async_copy(cb_hbm.at[idx_v], rows_v, sem).wait()
        pltpu.sync_copy(rows_v, out_hbm.at[pl.ds(base, _BPW)])

    return _sc_gather


def _fill_body(dmin_ref, ent_ref, q_ref, y_ref, loss_ref, err_ref):
    c = (jnp.sum(dmin_ref[...]) + 0.1 * jnp.sum(ent_ref[...])) / N
    loss_ref[...] = jnp.full((BF, N), c, dtype=jnp.float32)
    diff = q_ref[...] - y_ref[...]
    dev = jnp.sqrt(jnp.sum(diff * diff, axis=1))
    err_ref[...] = (dev > EPSILON).astype(jnp.float32)


_fill_call = pl.pallas_call(
    _fill_body,
    grid=(N // BF,),
    in_specs=[
        pl.BlockSpec((N,), lambda i: (0,)),
        pl.BlockSpec((N,), lambda i: (0,)),
        pl.BlockSpec((BF, D), lambda i: (i, 0)),
        pl.BlockSpec((BF, D), lambda i: (i, 0)),
    ],
    out_specs=[
        pl.BlockSpec((BF, N), lambda i: (i, 0)),
        pl.BlockSpec((BF,), lambda i: (i,)),
    ],
    out_shape=[
        jax.ShapeDtypeStruct((N, N), jnp.float32),
        jax.ShapeDtypeStruct((N,), jnp.float32),
    ],
)


def kernel(z, y_base, codebook, iter_k):
    idx, dmin, ent = _dist_call(z, codebook)
    quantized = _sc_gather_call()(codebook, idx)
    quant_loss, error_times = _fill_call(dmin, ent, quantized, y_base)
    return quantized, quant_loss, error_times
